# baseline (speedup 1.0000x reference)
import functools

import jax
import jax.numpy as jnp
import numpy as np
from jax import lax
from jax.experimental import pallas as pl
from jax.experimental.pallas import tpu as pltpu
from jax.experimental.pallas import tpu_sc as plsc

_HIDDEN = 256
_K = 300
_B = 8
_HW_SIZES = (128, 64, 32, 16)
_LEVEL_OFF = (0, 16384, 20480, 21504)
_ROWS_PER_BATCH = 21760
_NEG_INF = float("-inf")

_DIM_T = np.power(
    10000.0, 2.0 * np.floor(np.arange(128, dtype=np.float64) / 2.0) / 128.0
).astype(np.float32)


def _topk_kernel(lw_ref, hm_ref, coords_ref, idx_ref, w_ref, src_ref):
    b = pl.program_id(0)
    hm = hm_ref[0]

    def shift_col(a, d):
        z = jnp.zeros((256, abs(d)), jnp.float32)
        if d > 0:
            return jnp.concatenate([z, a[:, : 256 - d]], axis=1)
        return jnp.concatenate([a[:, -d:], z], axis=1)

    def shift_row(a, d):
        z = jnp.zeros((abs(d), 256), jnp.float32)
        if d > 0:
            return jnp.concatenate([z, a[: 256 - d, :]], axis=0)
        return jnp.concatenate([a[-d:, :], z], axis=0)

    cm = hm
    for d in (1, 2, -1, -2):
        cm = jnp.maximum(cm, shift_col(hm, d))
    pm = cm
    for d in (1, 2, -1, -2):
        pm = jnp.maximum(pm, shift_row(cm, d))

    keep = (hm == pm) & (hm > 0.1)
    cnt = jnp.sum(keep.astype(jnp.int32))
    masked = jnp.where(keep, hm, _NEG_INF)
    src = jnp.where(cnt < _K, hm, masked)
    src_ref[...] = src

    il256 = lax.broadcasted_iota(jnp.int32, (1, 256), 1)
    il300 = lax.broadcasted_iota(jnp.int32, (1, _K), 1)
    rowmax = jnp.max(src, axis=1).reshape(1, 256)

    def body(i, carry):
        rmax, xs, ys = carry
        m = jnp.max(rmax)
        r = jnp.min(jnp.where(rmax == m, il256, 65536))
        row = src_ref[pl.ds(r, 1), :]
        c = jnp.min(jnp.where(row == m, il256, 65536))
        row_new = jnp.where(il256 == c, _NEG_INF, row)
        src_ref[pl.ds(r, 1), :] = row_new
        rmax = jnp.where(il256 == r, jnp.max(row_new), rmax)
        xs = jnp.where(il300 == i, c.astype(jnp.float32), xs)
        ys = jnp.where(il300 == i, r.astype(jnp.float32), ys)
        return rmax, xs, ys

    zeros300 = jnp.zeros((1, _K), jnp.float32)
    _, xs, ys = lax.fori_loop(0, _K, body, (rowmax, zeros300, zeros300))

    cx = jnp.clip(xs / 255.0, 0.0, 1.0)
    cy = jnp.clip(ys / 255.0, 0.0, 1.0)
    coords_ref[...] = jnp.stack([cx, cy], axis=-1)

    lw = lw_ref[...]
    e = jnp.exp(lw - jnp.max(lw))
    lwn = e / jnp.sum(e)
    lw16 = jnp.broadcast_to(lwn.reshape(1, 4, 1), (1, 4, 4)).reshape(1, 1, 16)

    idx_parts = []
    w_parts = []
    base_b = b * _ROWS_PER_BATCH
    for wl, off in zip(_HW_SIZES, _LEVEL_OFF):
        x = jnp.clip(cx * (wl - 1.0), 0.0, wl - 1.0)
        y = jnp.clip(cy * (wl - 1.0), 0.0, wl - 1.0)
        x0 = jnp.floor(x)
        y0 = jnp.floor(y)
        x1 = jnp.minimum(x0 + 1.0, wl - 1.0)
        y1 = jnp.minimum(y0 + 1.0, wl - 1.0)
        wx = x - x0
        wy = y - y0
        x0i = x0.astype(jnp.int32)
        x1i = x1.astype(jnp.int32)
        y0i = y0.astype(jnp.int32)
        y1i = y1.astype(jnp.int32)
        base = base_b + off
        idx_parts.append(
            jnp.stack(
                [
                    base + y0i * wl + x0i,
                    base + y0i * wl + x1i,
                    base + y1i * wl + x0i,
                    base + y1i * wl + x1i,
                ],
                axis=-1,
            )
        )
        w_parts.append(
            jnp.stack(
                [
                    (1.0 - wx) * (1.0 - wy),
                    wx * (1.0 - wy),
                    (1.0 - wx) * wy,
                    wx * wy,
                ],
                axis=-1,
            )
        )
    idx_ref[...] = jnp.concatenate(idx_parts, axis=-1)
    w_ref[...] = jnp.concatenate(w_parts, axis=-1) * lw16


@jax.jit
def _run_topk(level_weights, heatmap):
    return pl.pallas_call(
        _topk_kernel,
        grid=(_B,),
        in_specs=[
            pl.BlockSpec((1, 4), lambda b: (0, 0)),
            pl.BlockSpec((1, 256, 256), lambda b: (b, 0, 0)),
        ],
        out_specs=[
            pl.BlockSpec((1, _K, 2), lambda b: (b, 0, 0)),
            pl.BlockSpec((1, _K, 16), lambda b: (b, 0, 0)),
            pl.BlockSpec((1, _K, 16), lambda b: (b, 0, 0)),
        ],
        out_shape=[
            jax.ShapeDtypeStruct((_B, _K, 2), jnp.float32),
            jax.ShapeDtypeStruct((_B, _K, 16), jnp.int32),
            jax.ShapeDtypeStruct((_B, _K, 16), jnp.float32),
        ],
        scratch_shapes=[pltpu.VMEM((256, 256), jnp.float32)],
    )(level_weights, heatmap)


_N_GATHER = _B * _K * 16
_N_WORKERS = 32
_ROWS_PER_W = _N_GATHER // _N_WORKERS
_CHUNK = 240
_N_CHUNKS = _ROWS_PER_W // _CHUNK

_gather_fn = None


def _build_gather():
    mesh = plsc.VectorSubcoreMesh(core_axis_name="c", subcore_axis_name="s")

    @functools.partial(
        pl.kernel,
        mesh=mesh,
        out_type=jax.ShapeDtypeStruct((_N_GATHER, _HIDDEN), jnp.float32),
        scratch_types=[
            pltpu.VMEM((_CHUNK,), jnp.int32),
            pltpu.VMEM((_CHUNK, _HIDDEN), jnp.float32),
            pltpu.SemaphoreType.DMA,
        ],
    )
    def gather_k(table_hbm, idx_hbm, out_hbm, idx_v, rows_v, sem):
        wid = lax.axis_index("s") * 2 + lax.axis_index("c")
        base = wid * _ROWS_PER_W
        for ch in range(_N_CHUNKS):
            o = base + ch * _CHUNK
            pltpu.sync_copy(idx_hbm.at[pl.ds(o, _CHUNK)], idx_v)
            pltpu.async_copy(table_hbm.at[idx_v], rows_v, sem).wait()
            pltpu.sync_copy(rows_v, out_hbm.at[pl.ds(o, _CHUNK)])

    return gather_k


def _sc_gather(table, idx_flat):
    global _gather_fn
    if _gather_fn is None:
        _gather_fn = _build_gather()
    return _gather_fn(table, idx_flat)


def _ln(x, g, b):
    mu = jnp.mean(x, axis=-1, keepdims=True)
    var = jnp.mean((x - mu) ** 2, axis=-1, keepdims=True)
    return (x - mu) / jnp.sqrt(var + 1e-5) * g + b


def _mlp_kernel(
    g_ref, w_ref, coords_ref, dimt_ref,
    cw1t_ref, cb1_ref, cg1_ref, cbt1_ref,
    cw2t_ref, cb2_ref, pwt_ref, pb_ref, pg_ref, pbt_ref,
    out_ref,
):
    g = g_ref[0]
    w = w_ref[0]
    pooled = jnp.sum(g * w[:, :, None], axis=1)

    h = jnp.dot(pooled, cw1t_ref[...], preferred_element_type=jnp.float32)
    h = h + cb1_ref[...]
    h = _ln(h, cg1_ref[...], cbt1_ref[...])
    h = jnp.maximum(h, 0.0)
    content = jnp.dot(h, cw2t_ref[...], preferred_element_type=jnp.float32)
    content = content + cb2_ref[...]

    cs = coords_ref[0]
    two_pi = 2.0 * np.pi
    dim_t = dimt_ref[...]
    px = (cs[:, 0:1] * two_pi) / dim_t
    py = (cs[:, 1:2] * two_pi) / dim_t
    par = lax.broadcasted_iota(jnp.int32, (1, 128), 1) % 2 == 0
    ex = jnp.where(par, jnp.sin(px), jnp.cos(px))
    ey = jnp.where(par, jnp.sin(py), jnp.cos(py))
    pos = jnp.concatenate([ex, ey], axis=-1)

    p = jnp.dot(pos, pwt_ref[...], preferred_element_type=jnp.float32)
    p = p + pb_ref[...]
    p = _ln(p, pg_ref[...], pbt_ref[...])

    out_ref[0] = jnp.concatenate([content, p], axis=-1)


@jax.jit
def _run_mlp(g4, w16, coords, dimt, cw1t, cb1, cg1, cbt1, cw2t, cb2, pwt, pb, pg, pbt):
    full = lambda shape: pl.BlockSpec(shape, lambda b: tuple(0 for _ in shape))
    return pl.pallas_call(
        _mlp_kernel,
        grid=(_B,),
        in_specs=[
            pl.BlockSpec((1, _K, 16, _HIDDEN), lambda b: (b, 0, 0, 0)),
            pl.BlockSpec((1, _K, 16), lambda b: (b, 0, 0)),
            pl.BlockSpec((1, _K, 2), lambda b: (b, 0, 0)),
            full((1, 128)),
            full((_HIDDEN, _HIDDEN)),
            full((1, _HIDDEN)),
            full((1, _HIDDEN)),
            full((1, _HIDDEN)),
            full((_HIDDEN, _HIDDEN)),
            full((1, _HIDDEN)),
            full((_HIDDEN, _HIDDEN)),
            full((1, _HIDDEN)),
            full((1, _HIDDEN)),
            full((1, _HIDDEN)),
        ],
        out_specs=pl.BlockSpec((1, _K, 2 * _HIDDEN), lambda b: (b, 0, 0)),
        out_shape=jax.ShapeDtypeStruct((_B, _K, 2 * _HIDDEN), jnp.float32),
    )(g4, w16, coords, dimt, cw1t, cb1, cg1, cbt1, cw2t, cb2, pwt, pb, pg, pbt)


def kernel(heatmap, feat0, feat1, feat2, feat3, level_weights,
           cW1, cb1, cg1, cbt1, cW2, cb2, pW, pb, pg, pbt):
    parts = [
        f.transpose(0, 2, 3, 1).reshape(_B, -1, _HIDDEN)
        for f in (feat0, feat1, feat2, feat3)
    ]
    table = jnp.concatenate(parts, axis=1).reshape(_B * _ROWS_PER_BATCH, _HIDDEN)

    coords, idx16, w16 = _run_topk(level_weights.reshape(1, 4), heatmap)
    gathered = _sc_gather(table, idx16.reshape(-1))
    g4 = gathered.reshape(_B, _K, 16, _HIDDEN)

    r = lambda v: v.reshape(1, _HIDDEN)
    return _run_mlp(
        g4, w16, coords, jnp.asarray(_DIM_T).reshape(1, 128),
        cW1.T, r(cb1), r(cg1), r(cbt1),
        cW2.T, r(cb2),
        pW.T, r(pb), r(pg), r(pbt),
    )


# speedup vs baseline: 1.6024x; 1.6024x over previous
import functools

import jax
import jax.numpy as jnp
import numpy as np
from jax import lax
from jax.experimental import pallas as pl
from jax.experimental.pallas import tpu as pltpu
from jax.experimental.pallas import tpu_sc as plsc

_HIDDEN = 256
_K = 300
_B = 8
_HW_SIZES = (128, 64, 32, 16)
_LEVEL_OFF = (0, 16384, 20480, 21504)
_ROWS_PER_BATCH = 21760
_NEG_INF = float("-inf")

_DIM_T = np.power(
    10000.0, 2.0 * np.floor(np.arange(128, dtype=np.float64) / 2.0) / 128.0
).astype(np.float32)


def _sget(a, b):
    return jnp.max(lax.slice_in_dim(a, b, b + 1, axis=0))


def _topk_kernel(lw_ref, hm_ref, coords_ref, idx_ref, w_ref, src_ref):
    hm = hm_ref[...]

    def shift_col(a, d):
        z = jnp.zeros((_B, 256, abs(d)), jnp.float32)
        if d > 0:
            return jnp.concatenate([z, a[:, :, : 256 - d]], axis=2)
        return jnp.concatenate([a[:, :, -d:], z], axis=2)

    def shift_row(a, d):
        z = jnp.zeros((_B, abs(d), 256), jnp.float32)
        if d > 0:
            return jnp.concatenate([z, a[:, : 256 - d, :]], axis=1)
        return jnp.concatenate([a[:, -d:, :], z], axis=1)

    cm = hm
    for d in (1, 2, -1, -2):
        cm = jnp.maximum(cm, shift_col(hm, d))
    pm = cm
    for d in (1, 2, -1, -2):
        pm = jnp.maximum(pm, shift_row(cm, d))

    keep = (hm == pm) & (hm > 0.1)
    cnt = jnp.sum(
        jnp.sum(keep.astype(jnp.int32), axis=2, keepdims=True),
        axis=1, keepdims=True,
    )
    masked = jnp.where(keep, hm, _NEG_INF)
    src = jnp.where(cnt < _K, hm, masked)
    src_ref[...] = src

    il256 = lax.broadcasted_iota(jnp.int32, (1, 256), 1)
    il256b = lax.broadcasted_iota(jnp.int32, (_B, 256), 1)
    il300b = lax.broadcasted_iota(jnp.int32, (_B, _K), 1)
    rowmax = jnp.max(src, axis=2)

    def body(i, carry):
        rmax, xs, ys = carry
        m = jnp.max(rmax, axis=1, keepdims=True)
        r = jnp.min(jnp.where(rmax == m, il256b, 65536), axis=1, keepdims=True)
        cb_list = []
        nrm_list = []
        for b in range(_B):
            rb = _sget(r, b)
            mb = _sget(m, b)
            row = src_ref[b, pl.ds(rb, 1), :]
            c = jnp.min(jnp.where(row == mb, il256, 65536))
            row_new = jnp.where(il256 == c, _NEG_INF, row)
            src_ref[b, pl.ds(rb, 1), :] = row_new
            cb_list.append(c)
            nrm_list.append(jnp.max(row_new))
        cs = jnp.stack(cb_list).reshape(_B, 1)
        nrm = jnp.stack(nrm_list).reshape(_B, 1)
        rmax = jnp.where(il256b == r, nrm, rmax)
        xs = jnp.where(il300b == i, cs.astype(jnp.float32), xs)
        ys = jnp.where(il300b == i, r.astype(jnp.float32), ys)
        return rmax, xs, ys

    zeros300 = jnp.zeros((_B, _K), jnp.float32)
    _, xs, ys = lax.fori_loop(0, _K, body, (rowmax, zeros300, zeros300))

    cx = jnp.clip(xs / 255.0, 0.0, 1.0)
    cy = jnp.clip(ys / 255.0, 0.0, 1.0)
    coords_ref[...] = jnp.stack([cx, cy], axis=-1)

    lw = lw_ref[...]
    e = jnp.exp(lw - jnp.max(lw))
    lwn = e / jnp.sum(e)
    lw16 = jnp.broadcast_to(lwn.reshape(1, 4, 1), (1, 4, 4)).reshape(1, 1, 16)

    idx_parts = []
    w_parts = []
    base_b = _ROWS_PER_BATCH * lax.broadcasted_iota(jnp.int32, (_B, _K), 0)
    for wl, off in zip(_HW_SIZES, _LEVEL_OFF):
        x = jnp.clip(cx * (wl - 1.0), 0.0, wl - 1.0)
        y = jnp.clip(cy * (wl - 1.0), 0.0, wl - 1.0)
        x0 = jnp.floor(x)
        y0 = jnp.floor(y)
        x1 = jnp.minimum(x0 + 1.0, wl - 1.0)
        y1 = jnp.minimum(y0 + 1.0, wl - 1.0)
        wx = x - x0
        wy = y - y0
        x0i = x0.astype(jnp.int32)
        x1i = x1.astype(jnp.int32)
        y0i = y0.astype(jnp.int32)
        y1i = y1.astype(jnp.int32)
        base = base_b + off
        idx_parts.append(
            jnp.stack(
                [
                    base + y0i * wl + x0i,
                    base + y0i * wl + x1i,
                    base + y1i * wl + x0i,
                    base + y1i * wl + x1i,
                ],
                axis=-1,
            )
        )
        w_parts.append(
            jnp.stack(
                [
                    (1.0 - wx) * (1.0 - wy),
                    wx * (1.0 - wy),
                    (1.0 - wx) * wy,
                    wx * wy,
                ],
                axis=-1,
            )
        )
    idx_ref[...] = jnp.concatenate(idx_parts, axis=-1)
    w_ref[...] = jnp.concatenate(w_parts, axis=-1) * lw16


@jax.jit
def _run_topk(level_weights, heatmap):
    return pl.pallas_call(
        _topk_kernel,
        out_shape=[
            jax.ShapeDtypeStruct((_B, _K, 2), jnp.float32),
            jax.ShapeDtypeStruct((_B, _K, 16), jnp.int32),
            jax.ShapeDtypeStruct((_B, _K, 16), jnp.float32),
        ],
        scratch_shapes=[pltpu.VMEM((_B, 256, 256), jnp.float32)],
    )(level_weights, heatmap)


_N_GATHER = _B * _K * 16
_N_WORKERS = 32
_ROWS_PER_W = _N_GATHER // _N_WORKERS
_CHUNK = 240
_N_CHUNKS = _ROWS_PER_W // _CHUNK

_gather_fn = None


def _build_gather():
    mesh = plsc.VectorSubcoreMesh(core_axis_name="c", subcore_axis_name="s")

    @functools.partial(
        pl.kernel,
        mesh=mesh,
        out_type=jax.ShapeDtypeStruct((_N_GATHER, _HIDDEN), jnp.float32),
        scratch_types=[
            pltpu.VMEM((_CHUNK,), jnp.int32),
            pltpu.VMEM((_CHUNK, _HIDDEN), jnp.float32),
            pltpu.SemaphoreType.DMA,
        ],
    )
    def gather_k(table_hbm, idx_hbm, out_hbm, idx_v, rows_v, sem):
        wid = lax.axis_index("s") * 2 + lax.axis_index("c")
        base = wid * _ROWS_PER_W
        for ch in range(_N_CHUNKS):
            o = base + ch * _CHUNK
            pltpu.sync_copy(idx_hbm.at[pl.ds(o, _CHUNK)], idx_v)
            pltpu.async_copy(table_hbm.at[idx_v], rows_v, sem).wait()
            pltpu.sync_copy(rows_v, out_hbm.at[pl.ds(o, _CHUNK)])

    return gather_k


def _sc_gather(table, idx_flat):
    global _gather_fn
    if _gather_fn is None:
        _gather_fn = _build_gather()
    return _gather_fn(table, idx_flat)


def _ln(x, g, b):
    mu = jnp.mean(x, axis=-1, keepdims=True)
    var = jnp.mean((x - mu) ** 2, axis=-1, keepdims=True)
    return (x - mu) / jnp.sqrt(var + 1e-5) * g + b


def _mlp_kernel(
    g_ref, w_ref, coords_ref, dimt_ref,
    cw1t_ref, cb1_ref, cg1_ref, cbt1_ref,
    cw2t_ref, cb2_ref, pwt_ref, pb_ref, pg_ref, pbt_ref,
    out_ref,
):
    g = g_ref[0]
    w = w_ref[0]
    pooled = jnp.sum(g * w[:, :, None], axis=1)

    h = jnp.dot(pooled, cw1t_ref[...], preferred_element_type=jnp.float32)
    h = h + cb1_ref[...]
    h = _ln(h, cg1_ref[...], cbt1_ref[...])
    h = jnp.maximum(h, 0.0)
    content = jnp.dot(h, cw2t_ref[...], preferred_element_type=jnp.float32)
    content = content + cb2_ref[...]

    cs = coords_ref[0]
    two_pi = 2.0 * np.pi
    dim_t = dimt_ref[...]
    px = (cs[:, 0:1] * two_pi) / dim_t
    py = (cs[:, 1:2] * two_pi) / dim_t
    par = lax.broadcasted_iota(jnp.int32, (1, 128), 1) % 2 == 0
    ex = jnp.where(par, jnp.sin(px), jnp.cos(px))
    ey = jnp.where(par, jnp.sin(py), jnp.cos(py))
    pos = jnp.concatenate([ex, ey], axis=-1)

    p = jnp.dot(pos, pwt_ref[...], preferred_element_type=jnp.float32)
    p = p + pb_ref[...]
    p = _ln(p, pg_ref[...], pbt_ref[...])

    out_ref[0] = jnp.concatenate([content, p], axis=-1)


@jax.jit
def _run_mlp(g4, w16, coords, dimt, cw1t, cb1, cg1, cbt1, cw2t, cb2, pwt, pb, pg, pbt):
    full = lambda shape: pl.BlockSpec(shape, lambda b: tuple(0 for _ in shape))
    return pl.pallas_call(
        _mlp_kernel,
        grid=(_B,),
        in_specs=[
            pl.BlockSpec((1, _K, 16, _HIDDEN), lambda b: (b, 0, 0, 0)),
            pl.BlockSpec((1, _K, 16), lambda b: (b, 0, 0)),
            pl.BlockSpec((1, _K, 2), lambda b: (b, 0, 0)),
            full((1, 128)),
            full((_HIDDEN, _HIDDEN)),
            full((1, _HIDDEN)),
            full((1, _HIDDEN)),
            full((1, _HIDDEN)),
            full((_HIDDEN, _HIDDEN)),
            full((1, _HIDDEN)),
            full((_HIDDEN, _HIDDEN)),
            full((1, _HIDDEN)),
            full((1, _HIDDEN)),
            full((1, _HIDDEN)),
        ],
        out_specs=pl.BlockSpec((1, _K, 2 * _HIDDEN), lambda b: (b, 0, 0)),
        out_shape=jax.ShapeDtypeStruct((_B, _K, 2 * _HIDDEN), jnp.float32),
    )(g4, w16, coords, dimt, cw1t, cb1, cg1, cbt1, cw2t, cb2, pwt, pb, pg, pbt)


def kernel(heatmap, feat0, feat1, feat2, feat3, level_weights,
           cW1, cb1, cg1, cbt1, cW2, cb2, pW, pb, pg, pbt):
    parts = [
        f.transpose(0, 2, 3, 1).reshape(_B, -1, _HIDDEN)
        for f in (feat0, feat1, feat2, feat3)
    ]
    table = jnp.concatenate(parts, axis=1).reshape(_B * _ROWS_PER_BATCH, _HIDDEN)

    coords, idx16, w16 = _run_topk(level_weights.reshape(1, 4), heatmap)
    gathered = _sc_gather(table, idx16.reshape(-1))
    g4 = gathered.reshape(_B, _K, 16, _HIDDEN)

    r = lambda v: v.reshape(1, _HIDDEN)
    return _run_mlp(
        g4, w16, coords, jnp.asarray(_DIM_T).reshape(1, 128),
        cW1.T, r(cb1), r(cg1), r(cbt1),
        cW2.T, r(cb2),
        pW.T, r(pb), r(pg), r(pbt),
    )
